# Initial kernel scaffold; baseline (speedup 1.0000x reference)
import jax
import jax.numpy as jnp
from jax.experimental import pallas as pl


def kernel(x, edge_index, batch, W1, b1, W2, b2, W3, b3, Wl, bl):
    raise NotImplementedError("write your pallas kernel here")


# speedup vs baseline: 9.4916x; 9.4916x over previous
import functools

import jax
import jax.numpy as jnp
from jax import lax
from jax.experimental import pallas as pl
from jax.experimental.pallas import tpu as pltpu
from jax.experimental.pallas import tpu_sc as plsc

F32 = jnp.float32
I32 = jnp.int32
CHUNK = 128
NC = 2
NS = 16
LANES = 16


def _rup(x, m):
    return (x + m - 1) // m * m


def _zero_rows(ref, width):
    def row(i, _):
        for j in range(width // LANES):
            ref[i, pl.ds(j * LANES, LANES)] = jnp.zeros((LANES,), F32)
        return 0
    lax.fori_loop(0, CHUNK, row, 0)


@functools.lru_cache(maxsize=None)
def _build_stats(ep, npad, bpad):
    mesh = plsc.VectorSubcoreMesh(core_axis_name="c", subcore_axis_name="s")
    rpt = npad // NS
    nchunk_d = ep // (NC * NS * CHUNK)
    nchunk_b = bpad // (NC * NS * CHUNK)

    @functools.partial(
        pl.kernel, mesh=mesh,
        out_type=[jax.ShapeDtypeStruct((NC, npad, CHUNK), F32),
                  jax.ShapeDtypeStruct((NC, CHUNK, CHUNK), F32)],
        scratch_types=[pltpu.VMEM((CHUNK,), I32),
                       pltpu.VMEM((CHUNK, CHUNK), F32),
                       pltpu.VMEM((CHUNK, CHUNK), F32),
                       pltpu.VMEM_SHARED((npad, CHUNK), F32),
                       pltpu.VMEM_SHARED((CHUNK, CHUNK), F32)],
    )
    def k(dst_hbm, batch_hbm, deg_out, cnt_out, idxv, onesv, zerov,
          acc_deg, acc_cnt):
        c = lax.axis_index("c")
        s = lax.axis_index("s")
        wid = s * NC + c

        def fill(i, _):
            for j in range(CHUNK // LANES):
                onesv[i, pl.ds(j * LANES, LANES)] = jnp.ones((LANES,), F32)
                zerov[i, pl.ds(j * LANES, LANES)] = jnp.zeros((LANES,), F32)
            return 0
        lax.fori_loop(0, CHUNK, fill, 0)

        def zstep(kk, _):
            pltpu.sync_copy(zerov, acc_deg.at[pl.ds(s * rpt + kk * CHUNK, CHUNK)])
            return 0
        lax.fori_loop(0, rpt // CHUNK, zstep, 0)

        @pl.when(s == 0)
        def _():
            pltpu.sync_copy(zerov, acc_cnt)

        plsc.subcore_barrier()

        ew = nchunk_d * CHUNK
        def dstep(kk, _):
            pltpu.sync_copy(dst_hbm.at[pl.ds(wid * ew + kk * CHUNK, CHUNK)], idxv)
            pltpu.sync_copy(onesv, acc_deg.at[idxv], add=True)
            return 0
        lax.fori_loop(0, nchunk_d, dstep, 0)

        bw = nchunk_b * CHUNK
        def bstep(kk, _):
            pltpu.sync_copy(batch_hbm.at[pl.ds(wid * bw + kk * CHUNK, CHUNK)], idxv)
            pltpu.sync_copy(onesv, acc_cnt.at[idxv], add=True)
            return 0
        lax.fori_loop(0, nchunk_b, bstep, 0)

        plsc.subcore_barrier()

        def wstep(kk, _):
            r0 = s * rpt + kk * CHUNK
            pltpu.sync_copy(acc_deg.at[pl.ds(r0, CHUNK)],
                            deg_out.at[c, pl.ds(r0, CHUNK)])
            return 0
        lax.fori_loop(0, rpt // CHUNK, wstep, 0)

        @pl.when(s == 0)
        def _():
            pltpu.sync_copy(acc_cnt, cnt_out.at[c])

    return k


@functools.lru_cache(maxsize=None)
def _build_prop_edge(ep, npad, f):
    mesh = plsc.VectorSubcoreMesh(core_axis_name="c", subcore_axis_name="s")
    rpt = npad // NS
    nchunk = ep // (NC * NS * CHUNK)

    @functools.partial(
        pl.kernel, mesh=mesh,
        out_type=jax.ShapeDtypeStruct((NC, npad, f), F32),
        scratch_types=[pltpu.VMEM((CHUNK,), I32),
                       pltpu.VMEM((CHUNK,), I32),
                       pltpu.VMEM((CHUNK, f), F32),
                       pltpu.VMEM((CHUNK, f), F32),
                       pltpu.VMEM_SHARED((npad, f), F32),
                       pltpu.SemaphoreType.DMA],
    )
    def k(g_hbm, src_hbm, dst_hbm, out, idxs, idxd, rowsv, zerov, acc, sem):
        c = lax.axis_index("c")
        s = lax.axis_index("s")
        wid = s * NC + c

        _zero_rows(zerov, f)

        def zstep(kk, _):
            pltpu.sync_copy(zerov, acc.at[pl.ds(s * rpt + kk * CHUNK, CHUNK)])
            return 0
        lax.fori_loop(0, rpt // CHUNK, zstep, 0)

        plsc.subcore_barrier()

        ew = nchunk * CHUNK
        def step(kk, _):
            off = wid * ew + kk * CHUNK
            pltpu.sync_copy(src_hbm.at[pl.ds(off, CHUNK)], idxs)
            pltpu.sync_copy(dst_hbm.at[pl.ds(off, CHUNK)], idxd)
            pltpu.async_copy(g_hbm.at[idxs], rowsv, sem).wait()
            pltpu.sync_copy(rowsv, acc.at[idxd], add=True)
            return 0
        lax.fori_loop(0, nchunk, step, 0)

        plsc.subcore_barrier()

        def wstep(kk, _):
            r0 = s * rpt + kk * CHUNK
            pltpu.sync_copy(acc.at[pl.ds(r0, CHUNK)], out.at[c, pl.ds(r0, CHUNK)])
            return 0
        lax.fori_loop(0, rpt // CHUNK, wstep, 0)

    return k


@functools.lru_cache(maxsize=None)
def _build_prop_half(ep, npad, f):
    mesh = plsc.VectorSubcoreMesh(core_axis_name="c", subcore_axis_name="s")
    rpt = npad // NS
    nchunk = ep // (NS * CHUNK)

    @functools.partial(
        pl.kernel, mesh=mesh,
        out_type=jax.ShapeDtypeStruct((NC, npad, f), F32),
        scratch_types=[pltpu.VMEM((CHUNK,), I32),
                       pltpu.VMEM((CHUNK,), I32),
                       pltpu.VMEM((CHUNK, f), F32),
                       pltpu.VMEM((CHUNK, f), F32),
                       pltpu.VMEM_SHARED((npad, f), F32),
                       pltpu.SemaphoreType.DMA],
    )
    def k(ga_hbm, gb_hbm, src_hbm, dst_hbm, out, idxs, idxd, rowsv, zerov,
          acc, sem):
        c = lax.axis_index("c")
        s = lax.axis_index("s")

        _zero_rows(zerov, f)

        def zstep(kk, _):
            pltpu.sync_copy(zerov, acc.at[pl.ds(s * rpt + kk * CHUNK, CHUNK)])
            return 0
        lax.fori_loop(0, rpt // CHUNK, zstep, 0)

        plsc.subcore_barrier()

        ew = nchunk * CHUNK

        def run(g_hbm):
            def step(kk, _):
                off = s * ew + kk * CHUNK
                pltpu.sync_copy(src_hbm.at[pl.ds(off, CHUNK)], idxs)
                pltpu.sync_copy(dst_hbm.at[pl.ds(off, CHUNK)], idxd)
                pltpu.async_copy(g_hbm.at[idxs], rowsv, sem).wait()
                pltpu.sync_copy(rowsv, acc.at[idxd], add=True)
                return 0
            lax.fori_loop(0, nchunk, step, 0)

        @pl.when(c == 0)
        def _():
            run(ga_hbm)

        @pl.when(c == 1)
        def _():
            run(gb_hbm)

        plsc.subcore_barrier()

        def wstep(kk, _):
            r0 = s * rpt + kk * CHUNK
            pltpu.sync_copy(acc.at[pl.ds(r0, CHUNK)], out.at[c, pl.ds(r0, CHUNK)])
            return 0
        lax.fori_loop(0, rpt // CHUNK, wstep, 0)

    return k


@functools.lru_cache(maxsize=None)
def _build_pool(b, f):
    mesh = plsc.VectorSubcoreMesh(core_axis_name="c", subcore_axis_name="s")

    @functools.partial(
        pl.kernel, mesh=mesh,
        out_type=jax.ShapeDtypeStruct((b, f), F32),
        scratch_types=[pltpu.VMEM((b,), I32),
                       pltpu.VMEM((b, f), F32),
                       pltpu.VMEM((b, f), F32),
                       pltpu.VMEM((b, f), F32),
                       pltpu.VMEM((b, f), F32),
                       pltpu.VMEM((f,), F32),
                       pltpu.VMEM((b, f), F32),
                       pltpu.SemaphoreType.DMA],
    )
    def k(li_hbm, s3a_hbm, s3b_hbm, g3_hbm, dinv_hbm, b3_hbm, out,
          liv, r1, r2, r3, r4, b3v, mv, sem):
        c = lax.axis_index("c")
        s = lax.axis_index("s")

        @pl.when(jnp.logical_and(c == 0, s == 0))
        def _():
            pltpu.sync_copy(li_hbm, liv)
            pltpu.sync_copy(b3_hbm, b3v)
            pltpu.async_copy(s3a_hbm.at[liv], r1, sem).wait()
            pltpu.async_copy(s3b_hbm.at[liv], r2, sem).wait()
            pltpu.async_copy(g3_hbm.at[liv], r3, sem).wait()
            pltpu.async_copy(dinv_hbm.at[liv], r4, sem).wait()

            def row(i, _):
                for j in range(f // LANES):
                    d = pl.ds(j * LANES, LANES)
                    mv[i, d] = (r1[i, d] + r2[i, d] + r3[i, d]) * r4[i, d] + b3v[d]
                return 0
            lax.fori_loop(0, b, row, 0)
            pltpu.sync_copy(mv, out)

    return k


def _tc_prep(degp, cntp, x, nblk, blk):
    n, fin = x.shape
    npad = degp.shape[1]

    def body(degp_ref, cntp_ref, x_ref, dinv_ref, g1_ref, last_ref):
        dp = degp_ref[0] + degp_ref[1]
        deg = dp[:, 0:1] + 1.0
        dv = lax.rsqrt(deg)
        dvb = jnp.broadcast_to(dv, (blk, fin))
        dinv_ref[...] = dvb
        g1_ref[...] = x_ref[...] * dvb
        cp = cntp_ref[0] + cntp_ref[1]
        cnt = cp[0:64, 0:1]
        ri = lax.broadcasted_iota(I32, (64, 64), 0)
        ci = lax.broadcasted_iota(I32, (64, 64), 1)
        lmask = (ci <= ri).astype(F32)
        csum = jnp.dot(lmask, cnt, preferred_element_type=F32)
        lastf = jnp.where(cnt > 0.0, csum - 1.0, 0.0)
        last_ref[...] = lastf.astype(I32)

    return pl.pallas_call(
        body,
        grid=(nblk,),
        in_specs=[
            pl.BlockSpec((NC, blk, CHUNK), lambda i: (0, i, 0)),
            pl.BlockSpec((NC, CHUNK, CHUNK), lambda i: (0, 0, 0)),
            pl.BlockSpec((blk, fin), lambda i: (i, 0)),
        ],
        out_specs=[
            pl.BlockSpec((blk, fin), lambda i: (i, 0)),
            pl.BlockSpec((blk, fin), lambda i: (i, 0)),
            pl.BlockSpec((64, 1), lambda i: (0, 0)),
        ],
        out_shape=[
            jax.ShapeDtypeStruct((n, fin), F32),
            jax.ShapeDtypeStruct((n, fin), F32),
            jax.ShapeDtypeStruct((64, 1), I32),
        ],
    )(degp, cntp, x)


def _tc_layer1(s1, g1, dinv, w1, b1, nblk, blk):
    n, fin = g1.shape
    h = w1.shape[1]
    hh = h // 2

    def body(s1_ref, g1_ref, dinv_ref, w1_ref, b1_ref, g2a_ref, g2b_ref):
        aggm = (s1_ref[0] + s1_ref[1] + g1_ref[...]) * dinv_ref[...]
        hv = jax.nn.relu(jnp.dot(aggm, w1_ref[...],
                                 preferred_element_type=F32) + b1_ref[...])
        g2 = hv * dinv_ref[:, 0:1]
        g2a_ref[...] = g2[:, :hh]
        g2b_ref[...] = g2[:, hh:]

    return pl.pallas_call(
        body,
        grid=(nblk,),
        in_specs=[
            pl.BlockSpec((NC, blk, fin), lambda i: (0, i, 0)),
            pl.BlockSpec((blk, fin), lambda i: (i, 0)),
            pl.BlockSpec((blk, fin), lambda i: (i, 0)),
            pl.BlockSpec((fin, h), lambda i: (0, 0)),
            pl.BlockSpec((1, h), lambda i: (0, 0)),
        ],
        out_specs=[
            pl.BlockSpec((blk, hh), lambda i: (i, 0)),
            pl.BlockSpec((blk, hh), lambda i: (i, 0)),
        ],
        out_shape=[
            jax.ShapeDtypeStruct((n, hh), F32),
            jax.ShapeDtypeStruct((n, hh), F32),
        ],
    )(s1, g1, dinv, w1, b1)


def _tc_layer23(s2, g2a, g2b, dinv, w2, b2, w3, nblk, blk):
    n, hh = g2a.shape
    h = w2.shape[0]
    f3 = w3.shape[1]

    def body(s2_ref, g2a_ref, g2b_ref, dinv_ref, w2_ref, b2_ref, w3_ref,
             g3_ref):
        dvb = dinv_ref[...]
        ma = (s2_ref[0] + g2a_ref[...]) * dvb
        mb = (s2_ref[1] + g2b_ref[...]) * dvb
        m = jnp.concatenate([ma, mb], axis=1)
        h2 = jax.nn.relu(jnp.dot(m, w2_ref[...],
                                 preferred_element_type=F32) + b2_ref[...])
        g3_ref[...] = jnp.dot(h2, w3_ref[...],
                              preferred_element_type=F32) * dvb

    return pl.pallas_call(
        body,
        grid=(nblk,),
        in_specs=[
            pl.BlockSpec((NC, blk, hh), lambda i: (0, i, 0)),
            pl.BlockSpec((blk, hh), lambda i: (i, 0)),
            pl.BlockSpec((blk, hh), lambda i: (i, 0)),
            pl.BlockSpec((blk, hh), lambda i: (i, 0)),
            pl.BlockSpec((h, h), lambda i: (0, 0)),
            pl.BlockSpec((1, h), lambda i: (0, 0)),
            pl.BlockSpec((h, f3), lambda i: (0, 0)),
        ],
        out_specs=pl.BlockSpec((blk, f3), lambda i: (i, 0)),
        out_shape=jax.ShapeDtypeStruct((n, f3), F32),
    )(s2, g2a, g2b, dinv, w2, b2, w3)


def _tc_head(m, wl, bl):
    b, f = m.shape
    cpad = wl.shape[1]

    def body(m_ref, wl_ref, bl_ref, out_ref):
        out_ref[...] = jnp.dot(m_ref[...], wl_ref[...],
                               preferred_element_type=F32) + bl_ref[...]

    return pl.pallas_call(
        body,
        out_shape=jax.ShapeDtypeStruct((b, cpad), F32),
    )(m, wl, bl)


def kernel(x, edge_index, batch, W1, b1, W2, b2, W3, b3, Wl, bl):
    n, fin = x.shape
    e = edge_index.shape[1]
    h = W1.shape[1]
    hh = h // 2
    c = Wl.shape[1]
    nseg = 64

    npad = _rup(n, NS * CHUNK)
    ep = _rup(e, NC * NS * CHUNK)
    bpad = _rup(n, NC * NS * CHUNK)
    blk = 1000
    nblk = n // blk

    src_p = jnp.concatenate(
        [edge_index[0], jnp.zeros((ep - e,), I32)])
    dst_p = jnp.concatenate(
        [edge_index[1], jnp.full((ep - e,), n, I32)])
    batch_p = jnp.concatenate(
        [batch, jnp.full((bpad - n,), nseg, I32)])

    degp, cntp = _build_stats(ep, npad, bpad)(dst_p, batch_p)

    dinv, g1, last2 = _tc_prep(degp, cntp, x, nblk, blk)
    li = last2.reshape(nseg)

    s1 = _build_prop_edge(ep, npad, fin)(g1, src_p, dst_p)
    g2a, g2b = _tc_layer1(s1, g1, dinv, W1, b1.reshape(1, h), nblk, blk)

    s2 = _build_prop_half(ep, npad, hh)(g2a, g2b, src_p, dst_p)

    g3 = _tc_layer23(s2, g2a, g2b, dinv, W2, b2.reshape(1, h), W3, nblk, blk)

    s3 = _build_prop_edge(ep, npad, fin)(g3, src_p, dst_p)

    m = _build_pool(nseg, hh)(li, s3[0], s3[1], g3, dinv, b3)

    wl_p = jnp.pad(Wl, ((0, 0), (0, CHUNK - c)))
    bl_p = jnp.pad(bl, (0, CHUNK - c)).reshape(1, CHUNK)
    out = _tc_head(m, wl_p, bl_p)
    return out[:, :c]
